# baseline (speedup 1.0000x reference)
import jax
import jax.numpy as jnp
from jax.experimental import pallas as pl

_KPROB = 0.5
_SCALE = 1.0 / _KPROB
_BLOCK = 256 * 1024


def _dropout_body(v_ref, r_ref, o_ref):
    mask = jnp.floor(r_ref[...] + _KPROB)
    o_ref[...] = v_ref[...] * (mask * _SCALE)


def kernel(indices, values, rand_vals):
    nnz = values.shape[0]
    grid = pl.cdiv(nnz, _BLOCK)
    new_vals = pl.pallas_call(
        _dropout_body,
        grid=(grid,),
        in_specs=[
            pl.BlockSpec((_BLOCK,), lambda i: (i,)),
            pl.BlockSpec((_BLOCK,), lambda i: (i,)),
        ],
        out_specs=pl.BlockSpec((_BLOCK,), lambda i: (i,)),
        out_shape=jax.ShapeDtypeStruct((nnz,), jnp.float32),
    )(values, rand_vals)
    return indices, new_vals
